# baseline (speedup 1.0000x reference)
import functools

import jax
import jax.numpy as jnp
from jax import lax
from jax.experimental import pallas as pl
from jax.experimental.pallas import tpu as pltpu
from jax.experimental.pallas import tpu_sc as plsc

_N = 10000
_E = 320000
_G = 64
_NC = 2
_NS = 16
_L = 16
_NW = _NC * _NS
_NP = 10240

_EPT = _E // _NW
_CH = 100


def _sc_mesh():
    return plsc.VectorSubcoreMesh(core_axis_name="c", subcore_axis_name="s")


_SC_PARAMS = pltpu.CompilerParams(needs_layout_passes=False,
                                  use_tc_tiling_on_sc=False)


def _sc_deg(dst, ew):
    @functools.partial(
        pl.kernel,
        out_type=jax.ShapeDtypeStruct((_NW, _N), jnp.float32),
        mesh=_sc_mesh(),
        compiler_params=_SC_PARAMS,
        scratch_types=[
            pltpu.VMEM((_EPT,), jnp.int32),
            pltpu.VMEM((_EPT,), jnp.float32),
            pltpu.VMEM((_N,), jnp.float32),
        ],
    )
    def k(dst_hbm, ew_hbm, out_hbm, idx_v, val_v, acc_v):
        w = lax.axis_index("s") * _NC + lax.axis_index("c")

        def zbody(i, _):
            acc_v[pl.ds(i * _L, _L)] = jnp.zeros((_L,), jnp.float32)
            return 0
        lax.fori_loop(0, _N // _L, zbody, 0)

        base = w * _EPT
        pltpu.sync_copy(dst_hbm.at[pl.ds(base, _EPT)], idx_v)
        pltpu.sync_copy(ew_hbm.at[pl.ds(base, _EPT)], val_v)

        def ebody(i, _):
            sl = pl.ds(i * _L, _L)
            plsc.addupdate_scatter(acc_v, [idx_v[sl]], val_v[sl])
            return 0
        lax.fori_loop(0, _EPT // _L, ebody, 0)
        pltpu.sync_copy(acc_v, out_hbm.at[w])

    return k(dst, ew)


def _sc_gather_norm(embW, evp, dinv, batch, src, dst, ew):
    GC = 80
    RC = _NP // _NW // GC

    @functools.partial(
        pl.kernel,
        out_type=(jax.ShapeDtypeStruct((_NW, RC, GC, 64), jnp.float32),
                  jax.ShapeDtypeStruct((_E,), jnp.float32),
                  jax.ShapeDtypeStruct((_E,), jnp.int32)),
        mesh=_sc_mesh(),
        compiler_params=_SC_PARAMS,
        scratch_types=[
            pltpu.VMEM((RC, GC), jnp.int32),
            pltpu.VMEM((RC, GC, 64), jnp.float32),
            pltpu.VMEM((_N,), jnp.float32),
            pltpu.VMEM((_N,), jnp.int32),
            pltpu.VMEM((_EPT,), jnp.int32),
            pltpu.VMEM((_EPT,), jnp.int32),
            pltpu.VMEM((_EPT,), jnp.float32),
            pltpu.VMEM((_EPT,), jnp.float32),
            pltpu.VMEM((_EPT,), jnp.int32),
            pltpu.SemaphoreType.DMA,
        ],
    )
    def k(embw_hbm, evp_hbm, dinv_hbm, batch_hbm, src_hbm, dst_hbm, ew_hbm,
          d1_hbm, norm_hbm, bdst_hbm,
          ids_v, rows_v, dinv_v, batch_v, srcv, dstv, ewv, normv, bdstv, sem):
        w = lax.axis_index("s") * _NC + lax.axis_index("c")

        pltpu.sync_copy(evp_hbm.at[w], ids_v)
        for j in range(RC):
            pltpu.async_copy(embw_hbm.at[ids_v.at[j]], rows_v.at[j], sem).wait()
        pltpu.sync_copy(rows_v, d1_hbm.at[w])

        pltpu.sync_copy(dinv_hbm, dinv_v)
        pltpu.sync_copy(batch_hbm, batch_v)

        ebase = w * _EPT
        pltpu.sync_copy(src_hbm.at[pl.ds(ebase, _EPT)], srcv)
        pltpu.sync_copy(dst_hbm.at[pl.ds(ebase, _EPT)], dstv)
        pltpu.sync_copy(ew_hbm.at[pl.ds(ebase, _EPT)], ewv)

        def ebody(i, _):
            sl = pl.ds(i * _L, _L)
            s = srcv[sl]
            d = dstv[sl]
            a = plsc.load_gather(dinv_v, [s])
            b = plsc.load_gather(dinv_v, [d])
            normv[sl] = ewv[sl] * a * b
            bdstv[sl] = plsc.load_gather(batch_v, [d])
            return 0
        lax.fori_loop(0, _EPT // _L, ebody, 0)
        pltpu.sync_copy(normv, norm_hbm.at[pl.ds(ebase, _EPT)])
        pltpu.sync_copy(bdstv, bdst_hbm.at[pl.ds(ebase, _EPT)])

    return k(embW, evp, dinv, batch, src, dst, ew)


def _sc_edge_pass(tables, src2, sidx2, nrm2, M, Dc, edge_split):
    MPT = M // _NS
    ZR = MPT if MPT <= 25 else 25
    ZCOPIES = MPT // ZR
    BLK = 50
    NCHT = (_E // _NC if edge_split else _E) // _NS // _CH
    NBLK = NCHT // BLK
    NBUF = 3

    @functools.partial(
        pl.kernel,
        out_type=jax.ShapeDtypeStruct((_NC, M, Dc), jnp.float32),
        mesh=_sc_mesh(),
        compiler_params=_SC_PARAMS,
        scratch_types=[
            pltpu.VMEM((BLK, _CH), jnp.int32),
            pltpu.VMEM((BLK, _CH), jnp.int32),
            pltpu.VMEM((BLK, _CH), jnp.float32),
            pltpu.VMEM((NBUF, _CH, Dc), jnp.bfloat16),
            pltpu.VMEM((NBUF, _CH, Dc), jnp.float32),
            pltpu.VMEM((ZR, Dc), jnp.float32),
            pltpu.VMEM_SHARED((M, Dc), jnp.float32),
            pltpu.SemaphoreType.DMA,
            pltpu.SemaphoreType.DMA,
        ],
    )
    def k(table_hbm, src_hbm, sidx_hbm, nrm_hbm, out_hbm,
          srcv, dstv, nrmv, rows_bf, rows_f, zbuf_v, acc_sh, gsem, ssem):
        c = lax.axis_index("c")
        s = lax.axis_index("s")
        tbl = table_hbm.at[c] if not edge_split else table_hbm.at[0]

        def zfill(i, _):
            r = i // (Dc // _L)
            j = i % (Dc // _L)
            zbuf_v[r, pl.ds(j * _L, _L)] = jnp.zeros((_L,), jnp.float32)
            return 0
        lax.fori_loop(0, ZR * (Dc // _L), zfill, 0)
        for t in range(ZCOPIES):
            pltpu.sync_copy(zbuf_v, acc_sh.at[pl.ds(s * MPT + t * ZR, ZR)])
        plsc.subcore_barrier()

        def gather_start(kk):
            par = lax.rem(kk, NBUF)
            pltpu.make_async_copy(tbl.at[srcv.at[kk]], rows_bf.at[par],
                                  gsem).start()

        def gather_wait(kk):
            par = lax.rem(kk, NBUF)
            pltpu.make_async_copy(tbl.at[srcv.at[kk]], rows_bf.at[par],
                                  gsem).wait()

        def scatter_start(kk):
            par = lax.rem(kk, NBUF)
            pltpu.make_async_copy(rows_f.at[par], acc_sh.at[dstv.at[kk]],
                                  ssem).start(add=True)

        def scatter_wait(kk):
            par = lax.rem(kk, NBUF)
            pltpu.make_async_copy(rows_f.at[par], acc_sh.at[dstv.at[kk]],
                                  ssem).wait()

        def block(b, _):
            if edge_split:
                pltpu.sync_copy(src_hbm.at[c, s, b], srcv)
                pltpu.sync_copy(sidx_hbm.at[c, s, b], dstv)
                pltpu.sync_copy(nrm_hbm.at[c, s, b], nrmv)
            else:
                pltpu.sync_copy(src_hbm.at[s, b], srcv)
                pltpu.sync_copy(sidx_hbm.at[s, b], dstv)
                pltpu.sync_copy(nrm_hbm.at[s, b], nrmv)
            gather_start(0)

            def chunk(kk, _):
                par = lax.rem(kk, NBUF)

                @pl.when(kk >= 2)
                def _():
                    scatter_wait(kk - 2)

                @pl.when(kk + 1 < BLK)
                def _():
                    gather_start(kk + 1)
                gather_wait(kk)
                iv = jnp.full((_L,), kk, dtype=jnp.int32)

                def escale(e4, _):
                    for u in range(4):
                        e = e4 * 4 + u
                        ev = jnp.full((_L,), e, dtype=jnp.int32)
                        sv = plsc.load_gather(nrmv, [iv, ev])
                        for q in range(Dc // 32):
                            v = plsc.bitcast(rows_bf[par, e, pl.ds(q * 32, 32)],
                                             jnp.int32)
                            lo = plsc.bitcast(v << 16, jnp.float32) * sv
                            hi = plsc.bitcast(v & jnp.int32(-65536),
                                              jnp.float32) * sv
                            rows_f[par, e, pl.ds(q * 32, _L)] = lo
                            rows_f[par, e, pl.ds(q * 32 + _L, _L)] = hi
                    return 0
                lax.fori_loop(0, _CH // 4, escale, 0)
                scatter_start(kk)
                return 0
            lax.fori_loop(0, BLK, chunk, 0)
            scatter_wait(BLK - 2)
            scatter_wait(BLK - 1)
            return 0
        lax.fori_loop(0, NBLK, block, 0)
        plsc.subcore_barrier()
        pltpu.sync_copy(acc_sh.at[pl.ds(s * MPT, MPT)],
                        out_hbm.at[c, pl.ds(s * MPT, MPT)])

    return k(tables, src2, sidx2, nrm2)


def _tc_pre(x, W_event, emb_table, W_embed):
    def body(x_ref, we_ref, emb_ref, wemb_ref, z2_ref, embw_ref):
        f = jnp.where(x_ref[...] == -1.0, 0.0, x_ref[...])
        z2_ref[...] = jnp.dot(f, we_ref[...], preferred_element_type=jnp.float32)
        embw_ref[...] = jnp.dot(emb_ref[...], wemb_ref[...],
                                preferred_element_type=jnp.float32)

    return pl.pallas_call(
        body,
        out_shape=(jax.ShapeDtypeStruct((_N, 128), jnp.float32),
                   jax.ShapeDtypeStruct((1000, 64), jnp.float32)),
    )(x, W_event, emb_table, W_embed)


def _tc_dinv(degp):
    def body(degp_ref, dinv_ref, dinvsq_ref):
        deg = jnp.sum(degp_ref[...], axis=0, keepdims=True) + 1.0
        dinv = lax.rsqrt(deg)
        dinv_ref[...] = dinv
        dinvsq_ref[...] = dinv * dinv

    return pl.pallas_call(
        body,
        out_shape=(jax.ShapeDtypeStruct((1, _N), jnp.float32),
                   jax.ShapeDtypeStruct((1, _N), jnp.float32)),
    )(degp)


def _tc_mid(Tp, ZT, dinv2, b12h, W_concat):
    def body(tp_ref, zt_ref, d2_ref, b_ref, w_ref, z3_ref):
        xc0 = tp_ref[0] + d2_ref[...] * zt_ref[0] + b_ref[0]
        xc1 = tp_ref[1] + d2_ref[...] * zt_ref[1] + b_ref[1]
        z3_ref[...] = (
            jnp.dot(xc0, w_ref[:96], preferred_element_type=jnp.float32)
            + jnp.dot(xc1, w_ref[96:], preferred_element_type=jnp.float32))

    return pl.pallas_call(
        body,
        out_shape=jax.ShapeDtypeStruct((_N, 128), jnp.float32),
    )(Tp, ZT, dinv2, b12h, W_concat)


def _tc_final(TB, z3, dinv2, batch2d, b_concat, seqf, W_seq, b_seq,
              W_cp_g, W_cp_s, b_cp, W_cls, b_cls):
    def body(tb_ref, z3_ref, d2_ref, batch_ref, bc_ref, sf_ref, ws_ref, bs_ref,
             wg_ref, wsq_ref, bcp_ref, wc_ref, bcl_ref, out_ref):
        onehot = (batch_ref[...] == lax.broadcasted_iota(jnp.int32, (1, _G), 1)
                  ).astype(jnp.float32)
        counts = lax.dot_general(onehot, jnp.ones((_N, 1), jnp.float32),
                                 (((0,), (0,)), ((), ())),
                                 preferred_element_type=jnp.float32)
        dz3 = d2_ref[...] * z3_ref[...]
        selfsum = lax.dot_general(onehot, dz3, (((0,), (0,)), ((), ())),
                                  preferred_element_type=jnp.float32)
        sums = tb_ref[0] + tb_ref[1] + selfsum + counts * bc_ref[...]
        graph_emb = sums / jnp.maximum(counts, 1.0)
        seq_out = jnp.dot(sf_ref[...], ws_ref[...],
                          preferred_element_type=jnp.float32) + bs_ref[...]
        cat = (jnp.dot(graph_emb, wg_ref[...], preferred_element_type=jnp.float32)
               + jnp.dot(seq_out, wsq_ref[...], preferred_element_type=jnp.float32)
               + bcp_ref[...])
        out_ref[...] = jnp.dot(jax.nn.relu(cat), wc_ref[...],
                               preferred_element_type=jnp.float32) + bcl_ref[...]

    return pl.pallas_call(
        body,
        out_shape=jax.ShapeDtypeStruct((_G, 10), jnp.float32),
    )(TB, z3, dinv2, batch2d, b_concat, seqf, W_seq, b_seq,
      W_cp_g, W_cp_s, b_cp, W_cls, b_cls)


def kernel(x, event_ids, edge_index, edge_attr, batch, sequence_features,
           emb_table, W_embed, b_embed, W_event, b_event, W_concat, b_concat,
           W_seq, b_seq, W_cp, b_cp, W_cls, b_cls):
    src = edge_index[0]
    dst = edge_index[1]

    degp = _sc_deg(dst, edge_attr)
    z2, embW = _tc_pre(x, W_event, emb_table, W_embed)
    dinv_row, dinvsq_row = _tc_dinv(degp)
    dinv_flat = dinv_row.reshape(_N)
    dinv2col = dinvsq_row.reshape(_N, 1)

    evp = jnp.pad(jnp.squeeze(event_ids, -1), (0, _NP - _N)
                  ).reshape(_NW, _NP // _NW // 80, 80)
    d1p, norm, bdst = _sc_gather_norm(embW, evp, dinv_flat, batch, src, dst,
                                      edge_attr)
    d1 = d1p.reshape(_NP, 64)[:_N]

    NBLKA = _E // _NS // _CH // 50
    srcA = src.reshape(_NS, NBLKA, 50, _CH)
    dstA = dst.reshape(_NS, NBLKA, 50, _CH)
    nrmA = norm.reshape(_NS, NBLKA, 50, _CH)
    NBLKB = _E // _NC // _NS // _CH // 50
    srcB = src.reshape(_NC, _NS, NBLKB, 50, _CH)
    bdstB = bdst.reshape(_NC, _NS, NBLKB, 50, _CH)
    nrmB = norm.reshape(_NC, _NS, NBLKB, 50, _CH)

    ZT = jnp.stack([jnp.concatenate([d1, z2[:, :32]], axis=1),
                    z2[:, 32:]])

    def _pairperm(t):
        s = t.shape
        return (t.astype(jnp.bfloat16)
                .reshape(s[:-1] + (s[-1] // 32, 2, 16))
                .swapaxes(-1, -2).reshape(s))

    Tp = _sc_edge_pass(_pairperm(ZT), srcA, dstA, nrmA, _N, 96, False)

    b12h = jnp.concatenate([b_embed, b_event]).reshape(2, 1, 96)
    z3 = _tc_mid(Tp, ZT, dinv2col, b12h, W_concat)

    TBp = _sc_edge_pass(_pairperm(z3)[None], srcB, bdstB, nrmB, _G, 128, True)

    out = _tc_final(TBp, z3, dinv2col, batch[:, None], b_concat[None, :],
                    sequence_features, W_seq, b_seq,
                    W_cp[:128], W_cp[128:], b_cp[None, :], W_cls, b_cls[None, :])
    return out


# speedup vs baseline: 1.7602x; 1.7602x over previous
import functools

import jax
import jax.numpy as jnp
from jax import lax
from jax.experimental import pallas as pl
from jax.experimental.pallas import tpu as pltpu
from jax.experimental.pallas import tpu_sc as plsc

_N = 10000
_E = 320000
_G = 64
_NC = 2
_NS = 16
_L = 16
_NW = _NC * _NS
_NP = 10240

_EPT = _E // _NW
_CH = 125


def _sc_mesh():
    return plsc.VectorSubcoreMesh(core_axis_name="c", subcore_axis_name="s")


_SC_PARAMS = pltpu.CompilerParams(needs_layout_passes=False,
                                  use_tc_tiling_on_sc=False)


def _sc_deg(dst, ew):
    @functools.partial(
        pl.kernel,
        out_type=jax.ShapeDtypeStruct((_NW, _N), jnp.float32),
        mesh=_sc_mesh(),
        compiler_params=_SC_PARAMS,
        scratch_types=[
            pltpu.VMEM((_EPT,), jnp.int32),
            pltpu.VMEM((_EPT,), jnp.float32),
            pltpu.VMEM((_N,), jnp.float32),
        ],
    )
    def k(dst_hbm, ew_hbm, out_hbm, idx_v, val_v, acc_v):
        w = lax.axis_index("s") * _NC + lax.axis_index("c")

        def zbody(i, _):
            acc_v[pl.ds(i * _L, _L)] = jnp.zeros((_L,), jnp.float32)
            return 0
        lax.fori_loop(0, _N // _L, zbody, 0)

        base = w * _EPT
        pltpu.sync_copy(dst_hbm.at[pl.ds(base, _EPT)], idx_v)
        pltpu.sync_copy(ew_hbm.at[pl.ds(base, _EPT)], val_v)

        def ebody(i, _):
            sl = pl.ds(i * _L, _L)
            plsc.addupdate_scatter(acc_v, [idx_v[sl]], val_v[sl])
            return 0
        lax.fori_loop(0, _EPT // _L, ebody, 0)
        pltpu.sync_copy(acc_v, out_hbm.at[w])

    return k(dst, ew)


def _sc_gather_norm(embW, evp, dinv, batch, src, dst, ew):
    GC = 80
    RC = _NP // _NW // GC

    @functools.partial(
        pl.kernel,
        out_type=(jax.ShapeDtypeStruct((_NW, RC, GC, 64), jnp.float32),
                  jax.ShapeDtypeStruct((_E,), jnp.float32),
                  jax.ShapeDtypeStruct((_E,), jnp.int32)),
        mesh=_sc_mesh(),
        compiler_params=_SC_PARAMS,
        scratch_types=[
            pltpu.VMEM((RC, GC), jnp.int32),
            pltpu.VMEM((RC, GC, 64), jnp.float32),
            pltpu.VMEM((_N,), jnp.float32),
            pltpu.VMEM((_N,), jnp.int32),
            pltpu.VMEM((_EPT,), jnp.int32),
            pltpu.VMEM((_EPT,), jnp.int32),
            pltpu.VMEM((_EPT,), jnp.float32),
            pltpu.VMEM((_EPT,), jnp.float32),
            pltpu.VMEM((_EPT,), jnp.int32),
            pltpu.SemaphoreType.DMA,
        ],
    )
    def k(embw_hbm, evp_hbm, dinv_hbm, batch_hbm, src_hbm, dst_hbm, ew_hbm,
          d1_hbm, norm_hbm, bdst_hbm,
          ids_v, rows_v, dinv_v, batch_v, srcv, dstv, ewv, normv, bdstv, sem):
        w = lax.axis_index("s") * _NC + lax.axis_index("c")

        pltpu.sync_copy(evp_hbm.at[w], ids_v)
        for j in range(RC):
            pltpu.async_copy(embw_hbm.at[ids_v.at[j]], rows_v.at[j], sem).wait()
        pltpu.sync_copy(rows_v, d1_hbm.at[w])

        pltpu.sync_copy(dinv_hbm, dinv_v)
        pltpu.sync_copy(batch_hbm, batch_v)

        ebase = w * _EPT
        pltpu.sync_copy(src_hbm.at[pl.ds(ebase, _EPT)], srcv)
        pltpu.sync_copy(dst_hbm.at[pl.ds(ebase, _EPT)], dstv)
        pltpu.sync_copy(ew_hbm.at[pl.ds(ebase, _EPT)], ewv)

        def ebody(i, _):
            sl = pl.ds(i * _L, _L)
            s = srcv[sl]
            d = dstv[sl]
            a = plsc.load_gather(dinv_v, [s])
            b = plsc.load_gather(dinv_v, [d])
            normv[sl] = ewv[sl] * a * b
            bdstv[sl] = plsc.load_gather(batch_v, [d])
            return 0
        lax.fori_loop(0, _EPT // _L, ebody, 0)
        pltpu.sync_copy(normv, norm_hbm.at[pl.ds(ebase, _EPT)])
        pltpu.sync_copy(bdstv, bdst_hbm.at[pl.ds(ebase, _EPT)])

    return k(embW, evp, dinv, batch, src, dst, ew)


def _sc_edge_pass(tables, src2, sidx2, nrm2, M, Dc, edge_split):
    MPT = M // _NS
    ZR = MPT if MPT <= 25 else 25
    ZCOPIES = MPT // ZR
    BLK = 40
    NCHT = (_E // _NC if edge_split else _E) // _NS // _CH
    NBLK = NCHT // BLK
    NBUF = 4

    @functools.partial(
        pl.kernel,
        out_type=jax.ShapeDtypeStruct((_NC, M, Dc), jnp.float32),
        mesh=_sc_mesh(),
        compiler_params=_SC_PARAMS,
        scratch_types=[
            pltpu.VMEM((BLK, _CH), jnp.int32),
            pltpu.VMEM((BLK, _CH), jnp.int32),
            pltpu.VMEM((BLK, _CH), jnp.float32),
            pltpu.VMEM((NBUF, _CH, Dc), jnp.float32),
            pltpu.VMEM((ZR, Dc), jnp.float32),
            pltpu.VMEM_SHARED((M, Dc), jnp.float32),
            pltpu.SemaphoreType.DMA,
            pltpu.SemaphoreType.DMA,
        ],
    )
    def k(table_hbm, src_hbm, sidx_hbm, nrm_hbm, out_hbm,
          srcv, dstv, nrmv, rows_v, zbuf_v, acc_sh, gsem, ssem):
        c = lax.axis_index("c")
        s = lax.axis_index("s")
        tbl = table_hbm.at[c] if not edge_split else table_hbm.at[0]

        def zfill(i, _):
            r = i // (Dc // _L)
            j = i % (Dc // _L)
            zbuf_v[r, pl.ds(j * _L, _L)] = jnp.zeros((_L,), jnp.float32)
            return 0
        lax.fori_loop(0, ZR * (Dc // _L), zfill, 0)
        for t in range(ZCOPIES):
            pltpu.sync_copy(zbuf_v, acc_sh.at[pl.ds(s * MPT + t * ZR, ZR)])
        plsc.subcore_barrier()

        def gather_start(kk, par):
            pltpu.make_async_copy(tbl.at[srcv.at[kk]], rows_v.at[par],
                                  gsem).start()

        def gather_wait(kk, par):
            pltpu.make_async_copy(tbl.at[srcv.at[kk]], rows_v.at[par],
                                  gsem).wait()

        def scatter_start(kk, par):
            pltpu.make_async_copy(rows_v.at[par], acc_sh.at[dstv.at[kk]],
                                  ssem).start(add=True)

        def scatter_wait(kk, par):
            pltpu.make_async_copy(rows_v.at[par], acc_sh.at[dstv.at[kk]],
                                  ssem).wait()

        def block(b, _):
            if edge_split:
                pltpu.sync_copy(src_hbm.at[c, s, b], srcv)
                pltpu.sync_copy(sidx_hbm.at[c, s, b], dstv)
                pltpu.sync_copy(nrm_hbm.at[c, s, b], nrmv)
            else:
                pltpu.sync_copy(src_hbm.at[s, b], srcv)
                pltpu.sync_copy(sidx_hbm.at[s, b], dstv)
                pltpu.sync_copy(nrm_hbm.at[s, b], nrmv)
            gather_start(0, 0)

            def group(g, _):
                for u in range(NBUF):
                    kk = g * NBUF + u

                    @pl.when(kk >= 2)
                    def _():
                        scatter_wait(kk - 2, (u - 2) % NBUF)

                    @pl.when(kk + 1 < BLK)
                    def _():
                        gather_start(kk + 1, (u + 1) % NBUF)
                    gather_wait(kk, u)
                    iv = jnp.full((_L,), kk, dtype=jnp.int32)

                    def escale(e4, _):
                        for v in range(4):
                            e = e4 * 4 + v
                            ev = jnp.full((_L,), e, dtype=jnp.int32)
                            sv = plsc.load_gather(nrmv, [iv, ev])
                            for j in range(Dc // _L):
                                sl = pl.ds(j * _L, _L)
                                rows_v[u, e, sl] = rows_v[u, e, sl] * sv
                        return 0
                    lax.fori_loop(0, _CH // 4, escale, 0)
                    scatter_start(kk, u)
                return 0
            lax.fori_loop(0, BLK // NBUF, group, 0)
            scatter_wait(BLK - 2, (BLK - 2) % NBUF)
            scatter_wait(BLK - 1, (BLK - 1) % NBUF)
            return 0
        lax.fori_loop(0, NBLK, block, 0)
        plsc.subcore_barrier()
        pltpu.sync_copy(acc_sh.at[pl.ds(s * MPT, MPT)],
                        out_hbm.at[c, pl.ds(s * MPT, MPT)])

    return k(tables, src2, sidx2, nrm2)


def _tc_pre(x, W_event, emb_table, W_embed):
    def body(x_ref, we_ref, emb_ref, wemb_ref, z2_ref, embw_ref):
        f = jnp.where(x_ref[...] == -1.0, 0.0, x_ref[...])
        z2_ref[...] = jnp.dot(f, we_ref[...], preferred_element_type=jnp.float32)
        embw_ref[...] = jnp.dot(emb_ref[...], wemb_ref[...],
                                preferred_element_type=jnp.float32)

    return pl.pallas_call(
        body,
        out_shape=(jax.ShapeDtypeStruct((_N, 128), jnp.float32),
                   jax.ShapeDtypeStruct((1000, 64), jnp.float32)),
    )(x, W_event, emb_table, W_embed)


def _tc_dinv(degp):
    def body(degp_ref, dinv_ref, dinvsq_ref):
        deg = jnp.sum(degp_ref[...], axis=0, keepdims=True) + 1.0
        dinv = lax.rsqrt(deg)
        dinv_ref[...] = dinv
        dinvsq_ref[...] = dinv * dinv

    return pl.pallas_call(
        body,
        out_shape=(jax.ShapeDtypeStruct((1, _N), jnp.float32),
                   jax.ShapeDtypeStruct((1, _N), jnp.float32)),
    )(degp)


def _tc_mid(Tp, ZT, dinv2, b12h, W_concat):
    def body(tp_ref, zt_ref, d2_ref, b_ref, w_ref, z3_ref):
        xc0 = tp_ref[0] + d2_ref[...] * zt_ref[0] + b_ref[0]
        xc1 = tp_ref[1] + d2_ref[...] * zt_ref[1] + b_ref[1]
        z3_ref[...] = (
            jnp.dot(xc0, w_ref[:96], preferred_element_type=jnp.float32)
            + jnp.dot(xc1, w_ref[96:], preferred_element_type=jnp.float32))

    return pl.pallas_call(
        body,
        out_shape=jax.ShapeDtypeStruct((_N, 128), jnp.float32),
    )(Tp, ZT, dinv2, b12h, W_concat)


def _tc_final(TB, z3, dinv2, batch2d, b_concat, seqf, W_seq, b_seq,
              W_cp_g, W_cp_s, b_cp, W_cls, b_cls):
    def body(tb_ref, z3_ref, d2_ref, batch_ref, bc_ref, sf_ref, ws_ref, bs_ref,
             wg_ref, wsq_ref, bcp_ref, wc_ref, bcl_ref, out_ref):
        onehot = (batch_ref[...] == lax.broadcasted_iota(jnp.int32, (1, _G), 1)
                  ).astype(jnp.float32)
        counts = lax.dot_general(onehot, jnp.ones((_N, 1), jnp.float32),
                                 (((0,), (0,)), ((), ())),
                                 preferred_element_type=jnp.float32)
        dz3 = d2_ref[...] * z3_ref[...]
        selfsum = lax.dot_general(onehot, dz3, (((0,), (0,)), ((), ())),
                                  preferred_element_type=jnp.float32)
        sums = tb_ref[0] + tb_ref[1] + selfsum + counts * bc_ref[...]
        graph_emb = sums / jnp.maximum(counts, 1.0)
        seq_out = jnp.dot(sf_ref[...], ws_ref[...],
                          preferred_element_type=jnp.float32) + bs_ref[...]
        cat = (jnp.dot(graph_emb, wg_ref[...], preferred_element_type=jnp.float32)
               + jnp.dot(seq_out, wsq_ref[...], preferred_element_type=jnp.float32)
               + bcp_ref[...])
        out_ref[...] = jnp.dot(jax.nn.relu(cat), wc_ref[...],
                               preferred_element_type=jnp.float32) + bcl_ref[...]

    return pl.pallas_call(
        body,
        out_shape=jax.ShapeDtypeStruct((_G, 10), jnp.float32),
    )(TB, z3, dinv2, batch2d, b_concat, seqf, W_seq, b_seq,
      W_cp_g, W_cp_s, b_cp, W_cls, b_cls)


def kernel(x, event_ids, edge_index, edge_attr, batch, sequence_features,
           emb_table, W_embed, b_embed, W_event, b_event, W_concat, b_concat,
           W_seq, b_seq, W_cp, b_cp, W_cls, b_cls):
    src = edge_index[0]
    dst = edge_index[1]

    degp = _sc_deg(dst, edge_attr)
    z2, embW = _tc_pre(x, W_event, emb_table, W_embed)
    dinv_row, dinvsq_row = _tc_dinv(degp)
    dinv_flat = dinv_row.reshape(_N)
    dinv2col = dinvsq_row.reshape(_N, 1)

    evp = jnp.pad(jnp.squeeze(event_ids, -1), (0, _NP - _N)
                  ).reshape(_NW, _NP // _NW // 80, 80)
    d1p, norm, bdst = _sc_gather_norm(embW, evp, dinv_flat, batch, src, dst,
                                      edge_attr)
    d1 = d1p.reshape(_NP, 64)[:_N]

    NBLKA = _E // _NS // _CH // 40
    srcA = src.reshape(_NS, NBLKA, 40, _CH)
    dstA = dst.reshape(_NS, NBLKA, 40, _CH)
    nrmA = norm.reshape(_NS, NBLKA, 40, _CH)
    NBLKB = _E // _NC // _NS // _CH // 40
    srcB = src.reshape(_NC, _NS, NBLKB, 40, _CH)
    bdstB = bdst.reshape(_NC, _NS, NBLKB, 40, _CH)
    nrmB = norm.reshape(_NC, _NS, NBLKB, 40, _CH)

    ZT = jnp.stack([jnp.concatenate([d1, z2[:, :32]], axis=1),
                    z2[:, 32:]])

    Tp = _sc_edge_pass(ZT, srcA, dstA, nrmA, _N, 96, False)

    b12h = jnp.concatenate([b_embed, b_event]).reshape(2, 1, 96)
    z3 = _tc_mid(Tp, ZT, dinv2col, b12h, W_concat)

    TBp = _sc_edge_pass(z3[None], srcB, bdstB, nrmB, _G, 128, True)

    out = _tc_final(TBp, z3, dinv2col, batch[:, None], b_concat[None, :],
                    sequence_features, W_seq, b_seq,
                    W_cp[:128], W_cp[128:], b_cp[None, :], W_cls, b_cls[None, :])
    return out
